# baseline (speedup 1.0000x reference)
import functools

import jax
import jax.numpy as jnp
import numpy as np
from jax import lax
from jax.experimental import pallas as pl
from jax.experimental.pallas import tpu as pltpu
from jax.experimental.pallas import tpu_sc as plsc

D_MODEL = 1024
SEQ_LEN = 2048
HALF = D_MODEL // 2

_NP_SPECS = (
    (4096, 2048, -1),
    (2048, 2048, -2),
    (1024, 2048, -4),
    (512, 2048, -8),
    (512, -SEQ_LEN, -1),
)

_POOL_N = (4096, 2048, 1024)

_ROWS_PER_STEP = 512


def _pe_body(vals_ref, invf_ref, *out_refs, starts):
    i = pl.program_id(0)
    v = vals_ref[0, 0, :]
    invf = invf_ref[0, :]
    arg8 = v[:8][:, None] * invf[None, :]
    s8 = jnp.sin(arg8)
    c8 = jnp.cos(arg8)
    d = v[1:2] - v[0:1]
    rots = []
    n = 8
    while n < _ROWS_PER_STEP:
        rot = (n * d)[:, None] * invf[None, :]
        rots.append((n, jnp.sin(rot), jnp.cos(rot)))
        n *= 2
    for k, ref in enumerate(out_refs):
        lo, hi = starts[k], starts[k + 1]

        @pl.when((i >= lo) & (i < hi))
        def _():
            ref[0:8, :HALF] = s8
            ref[0:8, HALF:] = c8
            for n, rs, rc in rots:
                s = ref[0:n, :HALF]
                c = ref[0:n, HALF:]
                ref[n:2 * n, :HALF] = s * rc + c * rs
                ref[n:2 * n, HALF:] = c * rc - s * rs


def _build_nps(dtype):
    nblocks = [n // _ROWS_PER_STEP for (n, _, _) in _NP_SPECS]
    starts = [0]
    for nb in nblocks:
        starts.append(starts[-1] + nb)
    total_steps = starts[-1]

    vals = np.concatenate([
        first + step * np.arange(n, dtype=np.float32)
        for (n, first, step) in _NP_SPECS
    ]).reshape(total_steps, 1, _ROWS_PER_STEP)
    vals = jnp.asarray(vals, dtype=dtype)

    freq = jnp.arange(HALF, dtype=dtype)
    invf = (1.0 / (10000.0 ** (freq / HALF)))[None, :]

    out_shapes = [jax.ShapeDtypeStruct((n, D_MODEL), dtype) for (n, _, _) in _NP_SPECS]

    def out_map(k):
        lo, nb = starts[k], nblocks[k]
        return lambda i: (jnp.clip(i - lo, 0, nb - 1), 0)

    return pl.pallas_call(
        functools.partial(_pe_body, starts=tuple(starts)),
        grid=(total_steps,),
        in_specs=[
            pl.BlockSpec((1, 1, _ROWS_PER_STEP), lambda i: (i, 0, 0)),
            pl.BlockSpec((1, HALF), lambda i: (0, 0)),
        ],
        out_specs=[
            pl.BlockSpec((_ROWS_PER_STEP, D_MODEL), out_map(k))
            for k in range(len(_NP_SPECS))
        ],
        out_shape=out_shapes,
    )(vals, invf)


_NC, _NS = 2, 16
_NW = _NC * _NS
_NPW = [n // _NW for n in _POOL_N]
_CHUNK = 48


def _pool_chunks(total):
    out = []
    co = 0
    while co < total:
        cs = min(_CHUNK, total - co)
        out.append((co, cs))
        co += cs
    return out


def _sc_body(src0, src1, src2, tail_ref, *rest):
    srcs = (src0, src1, src2)
    outs = rest[:3]
    b0, b1, g0, g1, s0, s1 = rest[3:]
    bufs, gsems, ssems = [b0, b1], [g0, g1], [s0, s1]
    wid = lax.axis_index("s") * _NC + lax.axis_index("c")
    last = wid == _NW - 1

    def run(shorten):
        sched = []
        for k in range(3):
            npw = _NPW[k]
            woff = wid * npw
            chunks = _pool_chunks(npw - 1 if shorten else npw)
            for (co, cs) in chunks:
                sched.append((k, (woff + co + 1) * D_MODEL,
                              (woff + co) * D_MODEL, cs * D_MODEL))
        n = len(sched)

        def gstart(c):
            k, so, _, ce = sched[c]
            return pltpu.async_copy(
                srcs[k].at[pl.ds(so, ce)], bufs[c % 2].at[pl.ds(0, ce)],
                gsems[c % 2])

        def sstart(c):
            k, _, do, ce = sched[c]
            return pltpu.async_copy(
                bufs[c % 2].at[pl.ds(0, ce)], outs[k].at[pl.ds(do, ce)],
                ssems[c % 2])

        gd = [None] * n
        sd = [None] * n
        gd[0] = gstart(0)
        for c in range(n):
            gd[c].wait()
            if c + 1 < n:
                if c >= 1:
                    sd[c - 1].wait()
                gd[c + 1] = gstart(c + 1)
            sd[c] = sstart(c)
        if n >= 2:
            sd[n - 2].wait()
        sd[n - 1].wait()

    @pl.when(jnp.logical_not(last))
    def _():
        run(False)

    @pl.when(last)
    def _():
        run(True)
        pltpu.sync_copy(tail_ref.at[pl.ds(0, D_MODEL)],
                        bufs[0].at[pl.ds(0, D_MODEL)])
        for k in range(3):
            pltpu.sync_copy(
                bufs[0].at[pl.ds(0, D_MODEL)],
                outs[k].at[pl.ds((_POOL_N[k] - 1) * D_MODEL, D_MODEL)],
            )


def _sc_pools(np0, np1, np2, tail, dtype):
    mesh = plsc.VectorSubcoreMesh(
        core_axis_name="c", subcore_axis_name="s",
        num_cores=_NC, num_subcores=_NS)
    out_type = [jax.ShapeDtypeStruct((n * D_MODEL,), dtype) for n in _POOL_N]
    outs = pl.kernel(
        _sc_body,
        out_type,
        mesh=mesh,
        scratch_types=[
            pltpu.VMEM((_CHUNK * D_MODEL,), jnp.float32),
            pltpu.VMEM((_CHUNK * D_MODEL,), jnp.float32),
            pltpu.SemaphoreType.DMA,
            pltpu.SemaphoreType.DMA,
            pltpu.SemaphoreType.DMA,
            pltpu.SemaphoreType.DMA,
        ],
    )(np0.reshape(-1), np1.reshape(-1), np2.reshape(-1), tail.reshape(-1))
    return [o.reshape(n, D_MODEL) for o, n in zip(outs, _POOL_N)]


_TT_ROWS = 512


def _tt_body(row_ref, full_ref, ttm_ref, cls_ref):
    j = pl.program_id(0)
    b = pl.program_id(1)
    shape = (_TT_ROWS, SEQ_LEN)
    rows = jnp.broadcast_to(row_ref[0, 0, :][:, None], shape)
    cols = jnp.broadcast_to(full_ref[0, 0, :][None, :], shape)
    ttm_ref[0] = (rows == cols) | (rows == 2) | (cols == 2)

    @pl.when(b == 0)
    def _():
        ri = jax.lax.broadcasted_iota(jnp.int32, shape, 0)
        ci = jax.lax.broadcasted_iota(jnp.int32, shape, 1)
        cls_ref[...] = (((ri + j * _TT_ROWS) > 0) & (ci > 0)).astype(cls_ref.dtype)


def _build_ttm(token_type_ids, dtype):
    batch = token_type_ids.shape[0]
    ids3 = token_type_ids.reshape(batch, 1, SEQ_LEN)
    nj = SEQ_LEN // _TT_ROWS
    return pl.pallas_call(
        _tt_body,
        grid=(nj, batch),
        in_specs=[
            pl.BlockSpec((1, 1, _TT_ROWS), lambda j, b: (b, 0, j)),
            pl.BlockSpec((1, 1, SEQ_LEN), lambda j, b: (b, 0, 0)),
        ],
        out_specs=[
            pl.BlockSpec((1, _TT_ROWS, SEQ_LEN), lambda j, b: (b, j, 0)),
            pl.BlockSpec((_TT_ROWS, SEQ_LEN), lambda j, b: (j, 0)),
        ],
        out_shape=[
            jax.ShapeDtypeStruct((batch, SEQ_LEN, SEQ_LEN), jnp.bool_),
            jax.ShapeDtypeStruct((SEQ_LEN, SEQ_LEN), dtype),
        ],
    )(ids3, ids3)


def kernel(inputs_embeds, attention_mask, token_type_ids):
    dtype = inputs_embeds.dtype
    np0, np1, np2, np3, tail = _build_nps(dtype)
    pool1, pool2, pool3 = _sc_pools(np0, np1, np2, tail, dtype)
    ttm, cls_mask = _build_ttm(token_type_ids, dtype)
    return (np0, np1, pool1, np2, pool2, np3, pool3, ttm, attention_mask, cls_mask)


# speedup vs baseline: 1.4980x; 1.4980x over previous
import functools

import jax
import jax.numpy as jnp
import numpy as np
from jax import lax
from jax.experimental import pallas as pl
from jax.experimental.pallas import tpu as pltpu
from jax.experimental.pallas import tpu_sc as plsc

D_MODEL = 1024
SEQ_LEN = 2048
HALF = D_MODEL // 2

_NP_SPECS = (
    (4096, 2048, -1),
    (2048, 2048, -2),
    (1024, 2048, -4),
    (512, 2048, -8),
    (512, -2020, -1),
)

_POOL_N = (4096, 2048, 1024)

_ROWS_PER_STEP = 512


def _pe_body(vals_ref, invf_ref, *out_refs, starts):
    i = pl.program_id(0)
    v = vals_ref[0, 0, :]
    invf = invf_ref[0, :]
    arg8 = v[:8][:, None] * invf[None, :]
    s8 = jnp.sin(arg8)
    c8 = jnp.cos(arg8)
    d = v[1:2] - v[0:1]
    rots = []
    n = 8
    while n < _ROWS_PER_STEP:
        rot = (n * d)[:, None] * invf[None, :]
        rots.append((n, jnp.sin(rot), jnp.cos(rot)))
        n *= 2
    for k, ref in enumerate(out_refs):
        lo, hi = starts[k], starts[k + 1]

        @pl.when((i >= lo) & (i < hi))
        def _():
            ref[0:8, :HALF] = s8
            ref[0:8, HALF:] = c8
            for n, rs, rc in rots:
                s = ref[0:n, :HALF]
                c = ref[0:n, HALF:]
                ref[n:2 * n, :HALF] = s * rc + c * rs
                ref[n:2 * n, HALF:] = c * rc - s * rs


def _build_nps(dtype):
    nblocks = [n // _ROWS_PER_STEP for (n, _, _) in _NP_SPECS]
    starts = [0]
    for nb in nblocks:
        starts.append(starts[-1] + nb)
    total_steps = starts[-1]

    vals = np.concatenate([
        first + step * np.arange(n, dtype=np.float32)
        for (n, first, step) in _NP_SPECS
    ]).reshape(total_steps, 1, _ROWS_PER_STEP)
    vals = jnp.asarray(vals, dtype=dtype)

    freq = jnp.arange(HALF, dtype=dtype)
    invf = (1.0 / (10000.0 ** (freq / HALF)))[None, :]

    out_shapes = [jax.ShapeDtypeStruct((n, D_MODEL), dtype) for (n, _, _) in _NP_SPECS]

    def out_map(k):
        lo, nb = starts[k], nblocks[k]
        return lambda i: (jnp.clip(i - lo, 0, nb - 1), 0)

    return pl.pallas_call(
        functools.partial(_pe_body, starts=tuple(starts)),
        grid=(total_steps,),
        in_specs=[
            pl.BlockSpec((1, 1, _ROWS_PER_STEP), lambda i: (i, 0, 0)),
            pl.BlockSpec((1, HALF), lambda i: (0, 0)),
        ],
        out_specs=[
            pl.BlockSpec((_ROWS_PER_STEP, D_MODEL), out_map(k))
            for k in range(len(_NP_SPECS))
        ],
        out_shape=out_shapes,
    )(vals, invf)


_NC, _NS = 2, 16
_NW = _NC * _NS
_NPW = [n // _NW for n in _POOL_N]
_CHUNK = 48


def _pool_chunks(total):
    out = []
    co = 0
    while co < total:
        cs = min(_CHUNK, total - co)
        out.append((co, cs))
        co += cs
    return out


_SEG = [0]
for _n in _NPW:
    _SEG.append(_SEG[-1] + _n)
_TSEG = _SEG[-1]
_IDX_LEN = _TSEG + 24

_TAIL_IDX = [
    [21 + i for i in range(8)],
    [14 + 2 * i for i in range(8)],
    [4 * i for i in range(8)],
]

_IDX_ALL = np.concatenate(
    [np.arange(1, n + 1) for n in _POOL_N] + [np.asarray(sum(_TAIL_IDX, []))]
).astype(np.int32)
_IDX_BASE = [0]
for _n in _POOL_N:
    _IDX_BASE.append(_IDX_BASE[-1] + _n)


def _sc_body(src0, src1, src2, tail_ref, idx_ref, *rest):
    srcs = (src0, src1, src2)
    outs = rest[:3]
    idx_v, b0, b1, g0, g1, s0, s1 = rest[3:]
    bufs, gsems, ssems = [b0, b1], [g0, g1], [s0, s1]
    wid = lax.axis_index("s") * _NC + lax.axis_index("c")
    last = wid == _NW - 1

    for k in range(3):
        npw = _NPW[k]
        pltpu.sync_copy(
            idx_ref.at[pl.ds(_IDX_BASE[k] + wid * npw, npw)],
            idx_v.at[pl.ds(_SEG[k], npw)],
        )
    pltpu.sync_copy(idx_ref.at[pl.ds(_IDX_BASE[-1], 24)],
                    idx_v.at[pl.ds(_TSEG, 24)])

    sched = []
    for k in range(3):
        npw = _NPW[k]
        woff = wid * npw
        for (co, cs) in _pool_chunks(npw - 8):
            sched.append((k, co, woff + co, cs))
    n = len(sched)

    def gstart(c):
        k, lo, _, cs = sched[c]
        return pltpu.async_copy(
            srcs[k].at[idx_v.at[pl.ds(_SEG[k] + lo, cs)]],
            bufs[c % 2].at[pl.ds(0, cs)],
            gsems[c % 2])

    def sstart(c):
        k, _, do, cs = sched[c]
        return pltpu.async_copy(
            bufs[c % 2].at[pl.ds(0, cs)], outs[k].at[pl.ds(do, cs)],
            ssems[c % 2])

    gd = [None] * n
    sd = [None] * n
    gd[0] = gstart(0)
    for c in range(n):
        gd[c].wait()
        if c + 1 < n:
            if c >= 1:
                sd[c - 1].wait()
            gd[c + 1] = gstart(c + 1)
        sd[c] = sstart(c)
    if n >= 2:
        sd[n - 2].wait()
    sd[n - 1].wait()

    @pl.when(jnp.logical_not(last))
    def _():
        for k in range(3):
            npw = _NPW[k]
            pltpu.async_copy(
                srcs[k].at[idx_v.at[pl.ds(_SEG[k] + npw - 8, 8)]],
                bufs[0].at[pl.ds(0, 8)], gsems[0]).wait()
            pltpu.sync_copy(bufs[0].at[pl.ds(0, 8)],
                            outs[k].at[pl.ds(wid * npw + npw - 8, 8)])

    @pl.when(last)
    def _():
        for k in range(3):
            pltpu.async_copy(
                tail_ref.at[idx_v.at[pl.ds(_TSEG + 8 * k, 8)]],
                bufs[0].at[pl.ds(0, 8)], gsems[0]).wait()
            pltpu.sync_copy(bufs[0].at[pl.ds(0, 8)],
                            outs[k].at[pl.ds(_POOL_N[k] - 8, 8)])


def _sc_pools(np0, np1, np2, tail, dtype):
    mesh = plsc.VectorSubcoreMesh(
        core_axis_name="c", subcore_axis_name="s",
        num_cores=_NC, num_subcores=_NS)
    out_type = [jax.ShapeDtypeStruct((n, D_MODEL), dtype) for n in _POOL_N]
    return pl.kernel(
        _sc_body,
        out_type,
        mesh=mesh,
        scratch_types=[
            pltpu.VMEM((_IDX_LEN,), jnp.int32),
            pltpu.VMEM((_CHUNK, D_MODEL), jnp.float32),
            pltpu.VMEM((_CHUNK, D_MODEL), jnp.float32),
            pltpu.SemaphoreType.DMA,
            pltpu.SemaphoreType.DMA,
            pltpu.SemaphoreType.DMA,
            pltpu.SemaphoreType.DMA,
        ],
    )(np0, np1, np2, tail, jnp.asarray(_IDX_ALL))


_TT_ROWS = 512


def _tt_body(row_ref, full_ref, ttm_ref, cls_ref):
    j = pl.program_id(0)
    b = pl.program_id(1)
    shape = (_TT_ROWS, SEQ_LEN)
    rows = jnp.broadcast_to(row_ref[0, 0, :][:, None], shape)
    cols = jnp.broadcast_to(full_ref[0, 0, :][None, :], shape)
    ttm_ref[0] = (rows == cols) | (rows == 2) | (cols == 2)

    @pl.when(b == 0)
    def _():
        ri = jax.lax.broadcasted_iota(jnp.int32, shape, 0)
        ci = jax.lax.broadcasted_iota(jnp.int32, shape, 1)
        cls_ref[...] = (((ri + j * _TT_ROWS) > 0) & (ci > 0)).astype(cls_ref.dtype)


def _build_ttm(token_type_ids, dtype):
    batch = token_type_ids.shape[0]
    ids3 = token_type_ids.reshape(batch, 1, SEQ_LEN)
    nj = SEQ_LEN // _TT_ROWS
    return pl.pallas_call(
        _tt_body,
        grid=(nj, batch),
        in_specs=[
            pl.BlockSpec((1, 1, _TT_ROWS), lambda j, b: (b, 0, j)),
            pl.BlockSpec((1, 1, SEQ_LEN), lambda j, b: (b, 0, 0)),
        ],
        out_specs=[
            pl.BlockSpec((1, _TT_ROWS, SEQ_LEN), lambda j, b: (b, j, 0)),
            pl.BlockSpec((_TT_ROWS, SEQ_LEN), lambda j, b: (j, 0)),
        ],
        out_shape=[
            jax.ShapeDtypeStruct((batch, SEQ_LEN, SEQ_LEN), jnp.bool_),
            jax.ShapeDtypeStruct((SEQ_LEN, SEQ_LEN), dtype),
        ],
    )(ids3, ids3)


def kernel(inputs_embeds, attention_mask, token_type_ids):
    dtype = inputs_embeds.dtype
    np0, np1, np2, np3, tail = _build_nps(dtype)
    pool1, pool2, pool3 = _sc_pools(np0, np1, np2, tail, dtype)
    ttm, cls_mask = _build_ttm(token_type_ids, dtype)
    return (np0, np1, pool1, np2, pool2, np3, pool3, ttm, attention_mask, cls_mask)


# speedup vs baseline: 1.5024x; 1.0029x over previous
import functools

import jax
import jax.numpy as jnp
import numpy as np
from jax import lax
from jax.experimental import pallas as pl
from jax.experimental.pallas import tpu as pltpu
from jax.experimental.pallas import tpu_sc as plsc

D_MODEL = 1024
SEQ_LEN = 2048
HALF = D_MODEL // 2

_NP_SPECS = (
    (4096, 2048, -1),
    (2048, 2048, -2),
    (1024, 2048, -4),
    (512, 2048, -8),
    (512, -2020, -1),
)

_POOL_N = (4096, 2048, 1024)

_ROWS_PER_STEP = 512


def _pe_body(vals_ref, invf_ref, *out_refs, starts):
    i = pl.program_id(0)
    v = vals_ref[0, 0, :]
    invf = invf_ref[0, :]
    arg8 = v[:8][:, None] * invf[None, :]
    s8 = jnp.sin(arg8)
    c8 = jnp.cos(arg8)
    d = v[1:2] - v[0:1]
    rots = []
    n = 8
    while n < _ROWS_PER_STEP:
        rot = (n * d)[:, None] * invf[None, :]
        rots.append((n, jnp.sin(rot), jnp.cos(rot)))
        n *= 2
    for k, ref in enumerate(out_refs):
        lo, hi = starts[k], starts[k + 1]

        @pl.when((i >= lo) & (i < hi))
        def _():
            ref[0:8, :HALF] = s8
            ref[0:8, HALF:] = c8
            for n, rs, rc in rots:
                s = ref[0:n, :HALF]
                c = ref[0:n, HALF:]
                ref[n:2 * n, :HALF] = s * rc + c * rs
                ref[n:2 * n, HALF:] = c * rc - s * rs


def _build_nps(dtype):
    nblocks = [n // _ROWS_PER_STEP for (n, _, _) in _NP_SPECS]
    starts = [0]
    for nb in nblocks:
        starts.append(starts[-1] + nb)
    total_steps = starts[-1]

    vals = np.concatenate([
        first + step * np.arange(n, dtype=np.float32)
        for (n, first, step) in _NP_SPECS
    ]).reshape(total_steps, 1, _ROWS_PER_STEP)
    vals = jnp.asarray(vals, dtype=dtype)

    freq = jnp.arange(HALF, dtype=dtype)
    invf = (1.0 / (10000.0 ** (freq / HALF)))[None, :]

    out_shapes = [jax.ShapeDtypeStruct((n, D_MODEL), dtype) for (n, _, _) in _NP_SPECS]

    def out_map(k):
        lo, nb = starts[k], nblocks[k]
        return lambda i: (jnp.clip(i - lo, 0, nb - 1), 0)

    return pl.pallas_call(
        functools.partial(_pe_body, starts=tuple(starts)),
        grid=(total_steps,),
        in_specs=[
            pl.BlockSpec((1, 1, _ROWS_PER_STEP), lambda i: (i, 0, 0)),
            pl.BlockSpec((1, HALF), lambda i: (0, 0)),
        ],
        out_specs=[
            pl.BlockSpec((_ROWS_PER_STEP, D_MODEL), out_map(k))
            for k in range(len(_NP_SPECS))
        ],
        out_shape=out_shapes,
    )(vals, invf)


_NC, _NS = 2, 16
_NW = _NC * _NS
_NPW = [n // _NW for n in _POOL_N]
_CHUNK = 56


def _pool_chunks(total):
    out = []
    co = 0
    while co < total:
        cs = min(_CHUNK, total - co)
        out.append((co, cs))
        co += cs
    return out


_SEG = [0]
for _n in _NPW:
    _SEG.append(_SEG[-1] + _n)
_TSEG = _SEG[-1]
_IDX_LEN = _TSEG + 24

_TAIL_IDX = [
    [21 + i for i in range(8)],
    [14 + 2 * i for i in range(8)],
    [4 * i for i in range(8)],
]

_IDX_ALL = np.concatenate(
    [np.arange(1, n + 1) for n in _POOL_N] + [np.asarray(sum(_TAIL_IDX, []))]
).astype(np.int32)
_IDX_BASE = [0]
for _n in _POOL_N:
    _IDX_BASE.append(_IDX_BASE[-1] + _n)


def _sc_body(src0, src1, src2, tail_ref, idx_ref, *rest):
    srcs = (src0, src1, src2)
    outs = rest[:3]
    idx_v, b0, b1, g0, g1, s0, s1 = rest[3:]
    bufs, gsems, ssems = [b0, b1], [g0, g1], [s0, s1]
    wid = lax.axis_index("s") * _NC + lax.axis_index("c")
    last = wid == _NW - 1

    for k in range(3):
        npw = _NPW[k]
        pltpu.sync_copy(
            idx_ref.at[pl.ds(_IDX_BASE[k] + wid * npw, npw)],
            idx_v.at[pl.ds(_SEG[k], npw)],
        )
    pltpu.sync_copy(idx_ref.at[pl.ds(_IDX_BASE[-1], 24)],
                    idx_v.at[pl.ds(_TSEG, 24)])

    sched = []
    for k in range(3):
        npw = _NPW[k]
        woff = wid * npw
        for (co, cs) in _pool_chunks(npw - 8):
            sched.append((k, co, woff + co, cs))
    n = len(sched)

    def gstart(c):
        k, lo, _, cs = sched[c]
        return pltpu.async_copy(
            srcs[k].at[idx_v.at[pl.ds(_SEG[k] + lo, cs)]],
            bufs[c % 2].at[pl.ds(0, cs)],
            gsems[c % 2])

    def sstart(c):
        k, _, do, cs = sched[c]
        return pltpu.async_copy(
            bufs[c % 2].at[pl.ds(0, cs)], outs[k].at[pl.ds(do, cs)],
            ssems[c % 2])

    gd = [None] * n
    sd = [None] * n
    gd[0] = gstart(0)
    for c in range(n):
        gd[c].wait()
        if c + 1 < n:
            if c >= 1:
                sd[c - 1].wait()
            gd[c + 1] = gstart(c + 1)
        sd[c] = sstart(c)
    if n >= 2:
        sd[n - 2].wait()
    sd[n - 1].wait()

    @pl.when(jnp.logical_not(last))
    def _():
        for k in range(3):
            npw = _NPW[k]
            pltpu.async_copy(
                srcs[k].at[idx_v.at[pl.ds(_SEG[k] + npw - 8, 8)]],
                bufs[0].at[pl.ds(0, 8)], gsems[0]).wait()
            pltpu.sync_copy(bufs[0].at[pl.ds(0, 8)],
                            outs[k].at[pl.ds(wid * npw + npw - 8, 8)])

    @pl.when(last)
    def _():
        for k in range(3):
            pltpu.async_copy(
                tail_ref.at[idx_v.at[pl.ds(_TSEG + 8 * k, 8)]],
                bufs[0].at[pl.ds(0, 8)], gsems[0]).wait()
            pltpu.sync_copy(bufs[0].at[pl.ds(0, 8)],
                            outs[k].at[pl.ds(_POOL_N[k] - 8, 8)])


def _sc_pools(np0, np1, np2, tail, dtype):
    mesh = plsc.VectorSubcoreMesh(
        core_axis_name="c", subcore_axis_name="s",
        num_cores=_NC, num_subcores=_NS)
    out_type = [jax.ShapeDtypeStruct((n, D_MODEL), dtype) for n in _POOL_N]
    return pl.kernel(
        _sc_body,
        out_type,
        mesh=mesh,
        scratch_types=[
            pltpu.VMEM((_IDX_LEN,), jnp.int32),
            pltpu.VMEM((_CHUNK, D_MODEL), jnp.float32),
            pltpu.VMEM((_CHUNK, D_MODEL), jnp.float32),
            pltpu.SemaphoreType.DMA,
            pltpu.SemaphoreType.DMA,
            pltpu.SemaphoreType.DMA,
            pltpu.SemaphoreType.DMA,
        ],
    )(np0, np1, np2, tail, jnp.asarray(_IDX_ALL))


_TT_ROWS = 512


def _tt_body(row_ref, full_ref, ttm_ref, cls_ref):
    j = pl.program_id(0)
    b = pl.program_id(1)
    shape = (_TT_ROWS, SEQ_LEN)
    rows = jnp.broadcast_to(row_ref[0, 0, :][:, None], shape)
    cols = jnp.broadcast_to(full_ref[0, 0, :][None, :], shape)
    ttm_ref[0] = (rows == cols) | (rows == 2) | (cols == 2)

    @pl.when(b == 0)
    def _():
        ri = jax.lax.broadcasted_iota(jnp.int32, shape, 0)
        ci = jax.lax.broadcasted_iota(jnp.int32, shape, 1)
        cls_ref[...] = (((ri + j * _TT_ROWS) > 0) & (ci > 0)).astype(cls_ref.dtype)


def _build_ttm(token_type_ids, dtype):
    batch = token_type_ids.shape[0]
    ids3 = token_type_ids.reshape(batch, 1, SEQ_LEN)
    nj = SEQ_LEN // _TT_ROWS
    return pl.pallas_call(
        _tt_body,
        grid=(nj, batch),
        in_specs=[
            pl.BlockSpec((1, 1, _TT_ROWS), lambda j, b: (b, 0, j)),
            pl.BlockSpec((1, 1, SEQ_LEN), lambda j, b: (b, 0, 0)),
        ],
        out_specs=[
            pl.BlockSpec((1, _TT_ROWS, SEQ_LEN), lambda j, b: (b, j, 0)),
            pl.BlockSpec((_TT_ROWS, SEQ_LEN), lambda j, b: (j, 0)),
        ],
        out_shape=[
            jax.ShapeDtypeStruct((batch, SEQ_LEN, SEQ_LEN), jnp.bool_),
            jax.ShapeDtypeStruct((SEQ_LEN, SEQ_LEN), dtype),
        ],
    )(ids3, ids3)


def kernel(inputs_embeds, attention_mask, token_type_ids):
    dtype = inputs_embeds.dtype
    np0, np1, np2, np3, tail = _build_nps(dtype)
    pool1, pool2, pool3 = _sc_pools(np0, np1, np2, tail, dtype)
    ttm, cls_mask = _build_ttm(token_type_ids, dtype)
    return (np0, np1, pool1, np2, pool2, np3, pool3, ttm, attention_mask, cls_mask)


# speedup vs baseline: 1.5038x; 1.0010x over previous
import functools

import jax
import jax.numpy as jnp
import numpy as np
from jax import lax
from jax.experimental import pallas as pl
from jax.experimental.pallas import tpu as pltpu
from jax.experimental.pallas import tpu_sc as plsc

D_MODEL = 1024
SEQ_LEN = 2048
HALF = D_MODEL // 2

_NP_SPECS = (
    (4096, 2048, -1),
    (2048, 2048, -2),
    (1024, 2048, -4),
    (512, 2048, -8),
    (512, -2020, -1),
)

_POOL_N = (4096, 2048, 1024)

_ROWS_PER_STEP = 512


def _pe_body(vals_ref, invf_ref, *out_refs, starts):
    i = pl.program_id(0)
    v = vals_ref[0, 0, :]
    invf = invf_ref[0, :]
    arg8 = v[:8][:, None] * invf[None, :]
    s8 = jnp.sin(arg8)
    c8 = jnp.cos(arg8)
    d = v[1:2] - v[0:1]
    rots = []
    n = 8
    while n < _ROWS_PER_STEP:
        rot = (n * d)[:, None] * invf[None, :]
        rots.append((n, jnp.sin(rot), jnp.cos(rot)))
        n *= 2
    for k, ref in enumerate(out_refs):
        lo, hi = starts[k], starts[k + 1]

        @pl.when((i >= lo) & (i < hi))
        def _():
            ref[0:8, :HALF] = s8
            ref[0:8, HALF:] = c8
            for n, rs, rc in rots:
                s = ref[0:n, :HALF]
                c = ref[0:n, HALF:]
                ref[n:2 * n, :HALF] = s * rc + c * rs
                ref[n:2 * n, HALF:] = c * rc - s * rs


def _build_nps(dtype):
    nblocks = [n // _ROWS_PER_STEP for (n, _, _) in _NP_SPECS]
    starts = [0]
    for nb in nblocks:
        starts.append(starts[-1] + nb)
    total_steps = starts[-1]

    vals = np.concatenate([
        first + step * np.arange(n, dtype=np.float32)
        for (n, first, step) in _NP_SPECS
    ]).reshape(total_steps, 1, _ROWS_PER_STEP)
    vals = jnp.asarray(vals, dtype=dtype)

    freq = jnp.arange(HALF, dtype=dtype)
    invf = (1.0 / (10000.0 ** (freq / HALF)))[None, :]

    out_shapes = [jax.ShapeDtypeStruct((n, D_MODEL), dtype) for (n, _, _) in _NP_SPECS]

    def out_map(k):
        lo, nb = starts[k], nblocks[k]
        return lambda i: (jnp.clip(i - lo, 0, nb - 1), 0)

    return pl.pallas_call(
        functools.partial(_pe_body, starts=tuple(starts)),
        grid=(total_steps,),
        in_specs=[
            pl.BlockSpec((1, 1, _ROWS_PER_STEP), lambda i: (i, 0, 0)),
            pl.BlockSpec((1, HALF), lambda i: (0, 0)),
        ],
        out_specs=[
            pl.BlockSpec((_ROWS_PER_STEP, D_MODEL), out_map(k))
            for k in range(len(_NP_SPECS))
        ],
        out_shape=out_shapes,
    )(vals, invf)


_NC, _NS = 2, 16
_NW = _NC * _NS
_NPW = [n // _NW for n in _POOL_N]
_CHUNK = 56


def _pool_chunks(total):
    out = []
    co = 0
    while co < total:
        cs = min(_CHUNK, total - co)
        out.append((co, cs))
        co += cs
    return out


_SEG = [0]
for _n in _NPW:
    _SEG.append(_SEG[-1] + _n)
_TSEG = _SEG[-1]
_IDX_LEN = _TSEG + 24

_TAIL_IDX = [
    [21 + i for i in range(8)],
    [14 + 2 * i for i in range(8)],
    [4 * i for i in range(8)],
]

_IDX_ALL = np.concatenate(
    [np.arange(1, n + 1) for n in _POOL_N] + [np.asarray(sum(_TAIL_IDX, []))]
).astype(np.int32)
_IDX_BASE = [0]
for _n in _POOL_N:
    _IDX_BASE.append(_IDX_BASE[-1] + _n)


def _sc_body(src0, src1, src2, tail_ref, idx_ref, *rest):
    srcs = (src0, src1, src2)
    outs = rest[:3]
    idx_v, b0, b1, g0, g1, s0, s1 = rest[3:]
    bufs, gsems, ssems = [b0, b1], [g0, g1], [s0, s1]
    wid = lax.axis_index("s") * _NC + lax.axis_index("c")
    last = wid == _NW - 1

    for k in range(3):
        npw = _NPW[k]
        pltpu.sync_copy(
            idx_ref.at[pl.ds(_IDX_BASE[k] + wid * npw, npw)],
            idx_v.at[pl.ds(_SEG[k], npw)],
        )
    pltpu.sync_copy(idx_ref.at[pl.ds(_IDX_BASE[-1], 24)],
                    idx_v.at[pl.ds(_TSEG, 24)])

    sched = []
    for k in range(3):
        npw = _NPW[k]
        woff = wid * npw
        for (co, cs) in _pool_chunks(npw - 8):
            sched.append((k, co, woff + co, cs))
    n = len(sched)

    def gstart(c):
        k, lo, _, cs = sched[c]
        return pltpu.async_copy(
            srcs[k].at[idx_v.at[pl.ds(_SEG[k] + lo, cs)]],
            bufs[c % 2].at[pl.ds(0, cs)],
            gsems[c % 2])

    def sstart(c):
        k, _, do, cs = sched[c]
        return pltpu.async_copy(
            bufs[c % 2].at[pl.ds(0, cs)], outs[k].at[pl.ds(do, cs)],
            ssems[c % 2])

    gd = [None] * n
    sd = [None] * n
    gd[0] = gstart(0)
    for c in range(n):
        gd[c].wait()
        if c + 1 < n:
            if c >= 1:
                sd[c - 1].wait()
            gd[c + 1] = gstart(c + 1)
        sd[c] = sstart(c)
    if n >= 2:
        sd[n - 2].wait()
    sd[n - 1].wait()

    @pl.when(jnp.logical_not(last))
    def _():
        for k in range(3):
            npw = _NPW[k]
            pltpu.async_copy(
                srcs[k].at[idx_v.at[pl.ds(_SEG[k] + npw - 8, 8)]],
                bufs[0].at[pl.ds(0, 8)], gsems[0]).wait()
            pltpu.sync_copy(bufs[0].at[pl.ds(0, 8)],
                            outs[k].at[pl.ds(wid * npw + npw - 8, 8)])

    @pl.when(last)
    def _():
        for k in range(3):
            pltpu.async_copy(
                tail_ref.at[idx_v.at[pl.ds(_TSEG + 8 * k, 8)]],
                bufs[0].at[pl.ds(0, 8)], gsems[0]).wait()
            pltpu.sync_copy(bufs[0].at[pl.ds(0, 8)],
                            outs[k].at[pl.ds(_POOL_N[k] - 8, 8)])


def _sc_pools(np0, np1, np2, tail, dtype):
    mesh = plsc.VectorSubcoreMesh(
        core_axis_name="c", subcore_axis_name="s",
        num_cores=_NC, num_subcores=_NS)
    out_type = [jax.ShapeDtypeStruct((n, D_MODEL), dtype) for n in _POOL_N]
    return pl.kernel(
        _sc_body,
        out_type,
        mesh=mesh,
        scratch_types=[
            pltpu.VMEM((_IDX_LEN,), jnp.int32),
            pltpu.VMEM((_CHUNK, D_MODEL), jnp.float32),
            pltpu.VMEM((_CHUNK, D_MODEL), jnp.float32),
            pltpu.SemaphoreType.DMA,
            pltpu.SemaphoreType.DMA,
            pltpu.SemaphoreType.DMA,
            pltpu.SemaphoreType.DMA,
        ],
    )(np0, np1, np2, tail, jnp.asarray(_IDX_ALL))


_TT_ROWS = 512


def _tt_body(row_ref, full_ref, ttm_ref, cls_ref):
    j = pl.program_id(0)
    b = pl.program_id(1)
    shape = (_TT_ROWS, SEQ_LEN)
    rows = jnp.broadcast_to(row_ref[0, 0, :][:, None], shape)
    cols = jnp.broadcast_to(full_ref[0, 0, :][None, :], shape)
    ttm_ref[0] = (rows == cols) | (rows == 2) | (cols == 2)

    @pl.when(b == 0)
    def _():
        ri = jax.lax.broadcasted_iota(jnp.int32, shape, 0)
        ci = jax.lax.broadcasted_iota(jnp.int32, shape, 1)
        cls_ref[...] = (((ri + j * _TT_ROWS) > 0) & (ci > 0)).astype(cls_ref.dtype)


def _build_ttm(token_type_ids, dtype):
    batch = token_type_ids.shape[0]
    ids3 = token_type_ids.reshape(batch, 1, SEQ_LEN)
    nj = SEQ_LEN // _TT_ROWS
    return pl.pallas_call(
        _tt_body,
        grid=(nj, batch),
        in_specs=[
            pl.BlockSpec((1, 1, _TT_ROWS), lambda j, b: (b, 0, j)),
            pl.BlockSpec((1, 1, SEQ_LEN), lambda j, b: (b, 0, 0)),
        ],
        out_specs=[
            pl.BlockSpec((1, _TT_ROWS, SEQ_LEN), lambda j, b: (b, j, 0)),
            pl.BlockSpec((_TT_ROWS, SEQ_LEN), lambda j, b: (j, 0)),
        ],
        out_shape=[
            jax.ShapeDtypeStruct((batch, SEQ_LEN, SEQ_LEN), jnp.bool_),
            jax.ShapeDtypeStruct((SEQ_LEN, SEQ_LEN), dtype),
        ],
    )(ids3, ids3)


def kernel(inputs_embeds, attention_mask, token_type_ids):
    dtype = inputs_embeds.dtype
    np0, np1, np2, np3, tail = _build_nps(dtype)
    ttm, cls_mask = _build_ttm(token_type_ids, dtype)
    pool1, pool2, pool3 = _sc_pools(np0, np1, np2, tail, dtype)
    return (np0, np1, pool1, np2, pool2, np3, pool3, ttm, attention_mask, cls_mask)
